# baseline (device time: 894629 ns/iter reference)
import numpy as np

import jax
import jax.numpy as jnp
from jax import lax
from jax.experimental import pallas as pl
from jax.experimental.pallas import tpu as pltpu

N_DEV = 8
SQ = 2048
SKV = 2048
D_MODEL = 1024
DH = 128
HEADS_PER_SHARD = 8
CHUNK = SQ // N_DEV
SCALE = 0.08838834764831843
NEG = -1e9

_blocks = np.arange(SQ // 64)
_order = np.concatenate(
    [_blocks[_blocks % 3 == 0], _blocks[_blocks % 3 == 1], _blocks[_blocks % 3 == 2]]
)
PERM = (_order[:, None] * 64 + np.arange(64)[None, :]).reshape(-1)
INV_PERM = np.argsort(PERM)
_n0 = int((_blocks % 3 == 0).sum()) * 64
_n1 = int((_blocks % 3 == 1).sum()) * 64
_n2 = int((_blocks % 3 == 2).sum()) * 64
REG = {0: (0, _n0), 1: (_n0, _n1), 2: (_n0 + _n1, _n2)}
CLS = [
    (REG[0], REG[0], None),
    (REG[1], REG[2], REG[1]),
    (REG[2], REG[1], REG[2]),
]


def _body(x_ref, wq_ref, k_hbm, v_hbm, wo_ref, out_ref,
          k_buf, v_buf, rs_recv, ag_recv,
          kv_sems, rs_ssem, rs_rsem, ag_ssem, ag_rsem):
    my = lax.axis_index("i")
    right = (my + 1) % N_DEV
    left = (my + N_DEV - 1) % N_DEV

    barrier_sem = pltpu.get_barrier_semaphore()
    for nbr in (left, right):
        pl.semaphore_signal(barrier_sem, inc=1, device_id=(nbr,),
                            device_id_type=pl.DeviceIdType.MESH)
    pl.semaphore_wait(barrier_sem, 2)

    for j in range(HEADS_PER_SHARD):
        c0, c1 = j * DH, (j + 1) * DH
        ck = pltpu.make_async_copy(k_hbm.at[:, c0:c1], k_buf, kv_sems.at[0])
        cv = pltpu.make_async_copy(v_hbm.at[:, c0:c1], v_buf, kv_sems.at[1])
        ck.start()
        cv.start()
        ck.wait()
        cv.wait()
        wq_j = wq_ref[:, c0:c1]
        wo_j = wo_ref[c0:c1, :]
        for (q0, qn), (a0, an), breg in CLS:
            q = jnp.dot(x_ref[q0:q0 + qn, :], wq_j,
                        preferred_element_type=jnp.float32)
            s_a = lax.dot_general(
                q, k_buf[a0:a0 + an, :], (((1,), (1,)), ((), ())),
                preferred_element_type=jnp.float32) * SCALE
            m_a = jnp.max(s_a, axis=-1, keepdims=True)
            if breg is None:
                w = jnp.exp(s_a - m_a)
                ctx = jnp.dot(w, v_buf[a0:a0 + an, :],
                              preferred_element_type=jnp.float32)
                ctx = ctx / jnp.sum(w, axis=-1, keepdims=True)
            else:
                b0, bn = breg
                s_b = lax.dot_general(
                    q, k_buf[b0:b0 + bn, :], (((1,), (1,)), ((), ())),
                    preferred_element_type=jnp.float32) * SCALE
                rb = lax.broadcasted_iota(jnp.int32, (qn, 1), 0) // 64
                cb = lax.broadcasted_iota(jnp.int32, (1, bn), 1) // 64
                s_b = jnp.where(rb == cb, s_b, NEG)
                s_c = lax.dot_general(
                    q, k_buf[0:64, :], (((1,), (1,)), ((), ())),
                    preferred_element_type=jnp.float32) * SCALE
                m = jnp.maximum(
                    m_a,
                    jnp.maximum(jnp.max(s_b, axis=-1, keepdims=True),
                                jnp.max(s_c, axis=-1, keepdims=True)))
                e_a = jnp.exp(s_a - m)
                e_b = jnp.exp(s_b - m)
                e_c = jnp.exp(s_c - m)
                denom = (jnp.sum(e_a, axis=-1, keepdims=True)
                         + jnp.sum(e_b, axis=-1, keepdims=True)
                         + jnp.sum(e_c, axis=-1, keepdims=True))
                ctx = (jnp.dot(e_a, v_buf[a0:a0 + an, :],
                               preferred_element_type=jnp.float32)
                       + jnp.dot(e_b, v_buf[b0:b0 + bn, :],
                                 preferred_element_type=jnp.float32)
                       + jnp.dot(e_c, v_buf[0:64, :],
                                 preferred_element_type=jnp.float32))
                ctx = ctx / denom
            part = jnp.dot(ctx, wo_j, preferred_element_type=jnp.float32)
            if j == 0:
                out_ref[q0:q0 + qn, :] = part
            else:
                out_ref[q0:q0 + qn, :] = out_ref[q0:q0 + qn, :] + part

    for h in range(N_DEV - 1):
        send_c = (my - h) % N_DEV
        recv_c = (my - h - 1) % N_DEV
        rdma = pltpu.make_async_remote_copy(
            src_ref=out_ref.at[pl.ds(send_c * CHUNK, CHUNK), :],
            dst_ref=rs_recv.at[h],
            send_sem=rs_ssem.at[h],
            recv_sem=rs_rsem.at[h],
            device_id=(right,),
            device_id_type=pl.DeviceIdType.MESH,
        )
        rdma.start()
        rdma.wait()
        out_ref[pl.ds(recv_c * CHUNK, CHUNK), :] = (
            out_ref[pl.ds(recv_c * CHUNK, CHUNK), :] + rs_recv[h])

    for h in range(N_DEV - 1):
        src = (out_ref.at[pl.ds(((my + 1) % N_DEV) * CHUNK, CHUNK), :]
               if h == 0 else ag_recv.at[h - 1])
        rdma = pltpu.make_async_remote_copy(
            src_ref=src,
            dst_ref=ag_recv.at[h],
            send_sem=ag_ssem.at[h],
            recv_sem=ag_rsem.at[h],
            device_id=(right,),
            device_id_type=pl.DeviceIdType.MESH,
        )
        rdma.start()
        rdma.wait()
        out_ref[pl.ds(((my - h) % N_DEV) * CHUNK, CHUNK), :] = ag_recv[h]


def kernel(x, Wq, K_ext, V_ext, Wo):
    pos = lax.axis_index("i")
    x2 = x.reshape(SQ, D_MODEL)[PERM]
    Ks = lax.dynamic_slice_in_dim(
        K_ext, pos * HEADS_PER_SHARD, HEADS_PER_SHARD, axis=2
    ).reshape(SKV, HEADS_PER_SHARD * DH)[PERM]
    Vs = lax.dynamic_slice_in_dim(
        V_ext, pos * HEADS_PER_SHARD, HEADS_PER_SHARD, axis=2
    ).reshape(SKV, HEADS_PER_SHARD * DH)[PERM]

    out = pl.pallas_call(
        _body,
        out_shape=jax.ShapeDtypeStruct((SQ, D_MODEL), jnp.float32),
        in_specs=[
            pl.BlockSpec(memory_space=pltpu.VMEM),
            pl.BlockSpec(memory_space=pltpu.VMEM),
            pl.BlockSpec(memory_space=pltpu.MemorySpace.HBM),
            pl.BlockSpec(memory_space=pltpu.MemorySpace.HBM),
            pl.BlockSpec(memory_space=pltpu.VMEM),
        ],
        out_specs=pl.BlockSpec(memory_space=pltpu.VMEM),
        scratch_shapes=[
            pltpu.VMEM((SKV, DH), jnp.float32),
            pltpu.VMEM((SKV, DH), jnp.float32),
            pltpu.VMEM((N_DEV - 1, CHUNK, D_MODEL), jnp.float32),
            pltpu.VMEM((N_DEV - 1, CHUNK, D_MODEL), jnp.float32),
            pltpu.SemaphoreType.DMA((2,)),
            pltpu.SemaphoreType.DMA((N_DEV - 1,)),
            pltpu.SemaphoreType.DMA((N_DEV - 1,)),
            pltpu.SemaphoreType.DMA((N_DEV - 1,)),
            pltpu.SemaphoreType.DMA((N_DEV - 1,)),
        ],
        compiler_params=pltpu.CompilerParams(
            collective_id=0,
            vmem_limit_bytes=63 * 1024 * 1024,
        ),
    )(x2, Wq, Ks, Vs, Wo)
    return out[INV_PERM].reshape(1, SQ, D_MODEL)


# device time: 316353 ns/iter; 2.8279x vs baseline; 2.8279x over previous
import numpy as np

import jax
import jax.numpy as jnp
from jax import lax
from jax.experimental import pallas as pl
from jax.experimental.pallas import tpu as pltpu

N_DEV = 8
SQ = 2048
SKV = 2048
D_MODEL = 1024
DH = 128
HEADS_PER_SHARD = 8
CHUNK = SQ // N_DEV
SCALE = 0.08838834764831843
NEG = -1e9

_blocks = np.arange(SQ // 64)
_n0 = int((_blocks % 3 == 0).sum()) * 64
_n1 = int((_blocks % 3 == 1).sum()) * 64
_n2 = int((_blocks % 3 == 2).sum()) * 64


def _permute_rows(a):
    a3 = a.reshape(SQ // 64, 64, a.shape[-1])
    return jnp.concatenate(
        [a3[0::3], a3[1::3], a3[2::3]], axis=0).reshape(SQ, a.shape[-1])


def _unpermute_rows(a):
    a3 = a.reshape(SQ // 64, 64, a.shape[-1])
    c0, c1, c2 = a3[0:11], a3[11:22], a3[22:32]
    c2p = jnp.concatenate([c2, jnp.zeros_like(a3[:1])], axis=0)
    inter = jnp.stack([c0, c1, c2p], axis=1).reshape(33, 64, a.shape[-1])
    return inter[:32].reshape(SQ, a.shape[-1])
REG = {0: (0, _n0), 1: (_n0, _n1), 2: (_n0 + _n1, _n2)}
CLS = [
    (REG[0], REG[0], None),
    (REG[1], REG[2], REG[1]),
    (REG[2], REG[1], REG[2]),
]


def _body(x_ref, wq_ref, k_hbm, v_hbm, wo_ref, out_ref,
          k_buf, v_buf, rs_recv, ag_recv,
          kv_sems, rs_ssem, rs_rsem, ag_ssem, ag_rsem):
    my = lax.axis_index("i")
    right = (my + 1) % N_DEV
    left = (my + N_DEV - 1) % N_DEV

    barrier_sem = pltpu.get_barrier_semaphore()
    for nbr in (left, right):
        pl.semaphore_signal(barrier_sem, inc=1, device_id=(nbr,),
                            device_id_type=pl.DeviceIdType.MESH)
    pl.semaphore_wait(barrier_sem, 2)

    for j in range(HEADS_PER_SHARD):
        c0, c1 = j * DH, (j + 1) * DH
        ck = pltpu.make_async_copy(k_hbm.at[:, c0:c1], k_buf, kv_sems.at[0])
        cv = pltpu.make_async_copy(v_hbm.at[:, c0:c1], v_buf, kv_sems.at[1])
        ck.start()
        cv.start()
        ck.wait()
        cv.wait()
        wq_j = wq_ref[:, c0:c1]
        wo_j = wo_ref[c0:c1, :]
        for (q0, qn), (a0, an), breg in CLS:
            q = jnp.dot(x_ref[q0:q0 + qn, :], wq_j,
                        preferred_element_type=jnp.float32)
            s_a = lax.dot_general(
                q, k_buf[a0:a0 + an, :], (((1,), (1,)), ((), ())),
                preferred_element_type=jnp.float32) * SCALE
            m_a = jnp.max(s_a, axis=-1, keepdims=True)
            if breg is None:
                w = jnp.exp(s_a - m_a)
                ctx = jnp.dot(w, v_buf[a0:a0 + an, :],
                              preferred_element_type=jnp.float32)
                ctx = ctx / jnp.sum(w, axis=-1, keepdims=True)
            else:
                b0, bn = breg
                s_b = lax.dot_general(
                    q, k_buf[b0:b0 + bn, :], (((1,), (1,)), ((), ())),
                    preferred_element_type=jnp.float32) * SCALE
                rb = lax.broadcasted_iota(jnp.int32, (qn, 1), 0) // 64
                cb = lax.broadcasted_iota(jnp.int32, (1, bn), 1) // 64
                s_b = jnp.where(rb == cb, s_b, NEG)
                s_c = lax.dot_general(
                    q, k_buf[0:64, :], (((1,), (1,)), ((), ())),
                    preferred_element_type=jnp.float32) * SCALE
                m = jnp.maximum(
                    m_a,
                    jnp.maximum(jnp.max(s_b, axis=-1, keepdims=True),
                                jnp.max(s_c, axis=-1, keepdims=True)))
                e_a = jnp.exp(s_a - m)
                e_b = jnp.exp(s_b - m)
                e_c = jnp.exp(s_c - m)
                denom = (jnp.sum(e_a, axis=-1, keepdims=True)
                         + jnp.sum(e_b, axis=-1, keepdims=True)
                         + jnp.sum(e_c, axis=-1, keepdims=True))
                ctx = (jnp.dot(e_a, v_buf[a0:a0 + an, :],
                               preferred_element_type=jnp.float32)
                       + jnp.dot(e_b, v_buf[b0:b0 + bn, :],
                                 preferred_element_type=jnp.float32)
                       + jnp.dot(e_c, v_buf[0:64, :],
                                 preferred_element_type=jnp.float32))
                ctx = ctx / denom
            part = jnp.dot(ctx, wo_j, preferred_element_type=jnp.float32)
            if j == 0:
                out_ref[q0:q0 + qn, :] = part
            else:
                out_ref[q0:q0 + qn, :] = out_ref[q0:q0 + qn, :] + part

    for h in range(N_DEV - 1):
        send_c = (my - h) % N_DEV
        recv_c = (my - h - 1) % N_DEV
        rdma = pltpu.make_async_remote_copy(
            src_ref=out_ref.at[pl.ds(send_c * CHUNK, CHUNK), :],
            dst_ref=rs_recv.at[h],
            send_sem=rs_ssem.at[h],
            recv_sem=rs_rsem.at[h],
            device_id=(right,),
            device_id_type=pl.DeviceIdType.MESH,
        )
        rdma.start()
        rdma.wait()
        out_ref[pl.ds(recv_c * CHUNK, CHUNK), :] = (
            out_ref[pl.ds(recv_c * CHUNK, CHUNK), :] + rs_recv[h])

    for h in range(N_DEV - 1):
        src = (out_ref.at[pl.ds(((my + 1) % N_DEV) * CHUNK, CHUNK), :]
               if h == 0 else ag_recv.at[h - 1])
        rdma = pltpu.make_async_remote_copy(
            src_ref=src,
            dst_ref=ag_recv.at[h],
            send_sem=ag_ssem.at[h],
            recv_sem=ag_rsem.at[h],
            device_id=(right,),
            device_id_type=pl.DeviceIdType.MESH,
        )
        rdma.start()
        rdma.wait()
        out_ref[pl.ds(((my - h) % N_DEV) * CHUNK, CHUNK), :] = ag_recv[h]


def kernel(x, Wq, K_ext, V_ext, Wo):
    pos = lax.axis_index("i")
    x2 = _permute_rows(x.reshape(SQ, D_MODEL))
    Ks = _permute_rows(lax.dynamic_slice_in_dim(
        K_ext, pos * HEADS_PER_SHARD, HEADS_PER_SHARD, axis=2
    ).reshape(SKV, HEADS_PER_SHARD * DH))
    Vs = _permute_rows(lax.dynamic_slice_in_dim(
        V_ext, pos * HEADS_PER_SHARD, HEADS_PER_SHARD, axis=2
    ).reshape(SKV, HEADS_PER_SHARD * DH))

    out = pl.pallas_call(
        _body,
        out_shape=jax.ShapeDtypeStruct((SQ, D_MODEL), jnp.float32),
        in_specs=[
            pl.BlockSpec(memory_space=pltpu.VMEM),
            pl.BlockSpec(memory_space=pltpu.VMEM),
            pl.BlockSpec(memory_space=pltpu.MemorySpace.HBM),
            pl.BlockSpec(memory_space=pltpu.MemorySpace.HBM),
            pl.BlockSpec(memory_space=pltpu.VMEM),
        ],
        out_specs=pl.BlockSpec(memory_space=pltpu.VMEM),
        scratch_shapes=[
            pltpu.VMEM((SKV, DH), jnp.float32),
            pltpu.VMEM((SKV, DH), jnp.float32),
            pltpu.VMEM((N_DEV - 1, CHUNK, D_MODEL), jnp.float32),
            pltpu.VMEM((N_DEV - 1, CHUNK, D_MODEL), jnp.float32),
            pltpu.SemaphoreType.DMA((2,)),
            pltpu.SemaphoreType.DMA((N_DEV - 1,)),
            pltpu.SemaphoreType.DMA((N_DEV - 1,)),
            pltpu.SemaphoreType.DMA((N_DEV - 1,)),
            pltpu.SemaphoreType.DMA((N_DEV - 1,)),
        ],
        compiler_params=pltpu.CompilerParams(
            collective_id=0,
            vmem_limit_bytes=63 * 1024 * 1024,
        ),
    )(x2, Wq, Ks, Vs, Wo)
    return _unpermute_rows(out).reshape(1, SQ, D_MODEL)


# device time: 295564 ns/iter; 3.0269x vs baseline; 1.0703x over previous
import numpy as np

import jax
import jax.numpy as jnp
from jax import lax
from jax.experimental import pallas as pl
from jax.experimental.pallas import tpu as pltpu

N_DEV = 8
SQ = 2048
SKV = 2048
D_MODEL = 1024
DH = 128
HEADS_PER_SHARD = 8
CHUNK = SQ // N_DEV
SEG = 4
SEGR = CHUNK // SEG
SCALE = 0.08838834764831843
NEG = -1e9

_blocks = np.arange(SQ // 64)
_n0 = int((_blocks % 3 == 0).sum()) * 64
_n1 = int((_blocks % 3 == 1).sum()) * 64
_n2 = int((_blocks % 3 == 2).sum()) * 64


def _permute_rows(a):
    a3 = a.reshape(SQ // 64, 64, a.shape[-1])
    return jnp.concatenate(
        [a3[0::3], a3[1::3], a3[2::3]], axis=0).reshape(SQ, a.shape[-1])


def _unpermute_rows(a):
    a3 = a.reshape(SQ // 64, 64, a.shape[-1])
    c0, c1, c2 = a3[0:11], a3[11:22], a3[22:32]
    c2p = jnp.concatenate([c2, jnp.zeros_like(a3[:1])], axis=0)
    inter = jnp.stack([c0, c1, c2p], axis=1).reshape(33, 64, a.shape[-1])
    return inter[:32].reshape(SQ, a.shape[-1])
REG = {0: (0, _n0), 1: (_n0, _n1), 2: (_n0 + _n1, _n2)}
CLS = [
    (REG[0], REG[0], None),
    (REG[1], REG[2], REG[1]),
    (REG[2], REG[1], REG[2]),
]


def _body(x_ref, wq_ref, k_hbm, v_hbm, wo_ref, out_ref,
          k_buf, v_buf, rs_recv, ag_recv,
          kv_sems, rs_ssem, rs_rsem, ag_ssem, ag_rsem):
    my = lax.axis_index("i")
    right = (my + 1) % N_DEV
    left = (my + N_DEV - 1) % N_DEV

    barrier_sem = pltpu.get_barrier_semaphore()
    for nbr in (left, right):
        pl.semaphore_signal(barrier_sem, inc=1, device_id=(nbr,),
                            device_id_type=pl.DeviceIdType.MESH)
    pl.semaphore_wait(barrier_sem, 2)

    for j in range(HEADS_PER_SHARD):
        c0, c1 = j * DH, (j + 1) * DH
        ck = pltpu.make_async_copy(k_hbm.at[:, c0:c1], k_buf, kv_sems.at[0])
        cv = pltpu.make_async_copy(v_hbm.at[:, c0:c1], v_buf, kv_sems.at[1])
        ck.start()
        cv.start()
        ck.wait()
        cv.wait()
        wq_j = wq_ref[:, c0:c1]
        wo_j = wo_ref[c0:c1, :]
        for (q0, qn), (a0, an), breg in CLS:
            q = jnp.dot(x_ref[q0:q0 + qn, :], wq_j,
                        preferred_element_type=jnp.float32)
            s_a = lax.dot_general(
                q, k_buf[a0:a0 + an, :], (((1,), (1,)), ((), ())),
                preferred_element_type=jnp.float32) * SCALE
            m_a = jnp.max(s_a, axis=-1, keepdims=True)
            if breg is None:
                w = jnp.exp(s_a - m_a)
                ctx = jnp.dot(w, v_buf[a0:a0 + an, :],
                              preferred_element_type=jnp.float32)
                ctx = ctx / jnp.sum(w, axis=-1, keepdims=True)
            else:
                b0, bn = breg
                s_b = lax.dot_general(
                    q, k_buf[b0:b0 + bn, :], (((1,), (1,)), ((), ())),
                    preferred_element_type=jnp.float32) * SCALE
                rb = lax.broadcasted_iota(jnp.int32, (qn, 1), 0) // 64
                cb = lax.broadcasted_iota(jnp.int32, (1, bn), 1) // 64
                s_b = jnp.where(rb == cb, s_b, NEG)
                s_c = lax.dot_general(
                    q, k_buf[0:64, :], (((1,), (1,)), ((), ())),
                    preferred_element_type=jnp.float32) * SCALE
                m = jnp.maximum(
                    m_a,
                    jnp.maximum(jnp.max(s_b, axis=-1, keepdims=True),
                                jnp.max(s_c, axis=-1, keepdims=True)))
                e_a = jnp.exp(s_a - m)
                e_b = jnp.exp(s_b - m)
                e_c = jnp.exp(s_c - m)
                denom = (jnp.sum(e_a, axis=-1, keepdims=True)
                         + jnp.sum(e_b, axis=-1, keepdims=True)
                         + jnp.sum(e_c, axis=-1, keepdims=True))
                ctx = (jnp.dot(e_a, v_buf[a0:a0 + an, :],
                               preferred_element_type=jnp.float32)
                       + jnp.dot(e_b, v_buf[b0:b0 + bn, :],
                                 preferred_element_type=jnp.float32)
                       + jnp.dot(e_c, v_buf[0:64, :],
                                 preferred_element_type=jnp.float32))
                ctx = ctx / denom
            part = jnp.dot(ctx, wo_j, preferred_element_type=jnp.float32)
            if j == 0:
                out_ref[q0:q0 + qn, :] = part
            else:
                out_ref[q0:q0 + qn, :] = out_ref[q0:q0 + qn, :] + part

    def rs_desc(h, s):
        send_c = (my - h) % N_DEV
        return pltpu.make_async_remote_copy(
            src_ref=out_ref.at[pl.ds(send_c * CHUNK + s * SEGR, SEGR), :],
            dst_ref=rs_recv.at[h, pl.ds(s * SEGR, SEGR), :],
            send_sem=rs_ssem.at[h, s],
            recv_sem=rs_rsem.at[h, s],
            device_id=(right,),
            device_id_type=pl.DeviceIdType.MESH,
        )

    rs_sends = []
    for s in range(SEG):
        d = rs_desc(0, s)
        d.start()
        rs_sends.append(d)
    for h in range(N_DEV - 1):
        recv_c = (my - h - 1) % N_DEV
        for s in range(SEG):
            rs_desc(h, s).wait_recv()
            rows = pl.ds(recv_c * CHUNK + s * SEGR, SEGR)
            out_ref[rows, :] = (
                out_ref[rows, :] + rs_recv[h, s * SEGR:(s + 1) * SEGR, :])
            if h < N_DEV - 2:
                d = rs_desc(h + 1, s)
                d.start()
                rs_sends.append(d)
    for d in rs_sends:
        d.wait_send()

    def ag_desc(h, s):
        if h == 0:
            src = out_ref.at[
                pl.ds(((my + 1) % N_DEV) * CHUNK + s * SEGR, SEGR), :]
        else:
            src = ag_recv.at[h - 1, pl.ds(s * SEGR, SEGR), :]
        return pltpu.make_async_remote_copy(
            src_ref=src,
            dst_ref=ag_recv.at[h, pl.ds(s * SEGR, SEGR), :],
            send_sem=ag_ssem.at[h, s],
            recv_sem=ag_rsem.at[h, s],
            device_id=(right,),
            device_id_type=pl.DeviceIdType.MESH,
        )

    ag_sends = []
    for s in range(SEG):
        d = ag_desc(0, s)
        d.start()
        ag_sends.append(d)
    for h in range(N_DEV - 1):
        for s in range(SEG):
            ag_desc(h, s).wait_recv()
            if h < N_DEV - 2:
                d = ag_desc(h + 1, s)
                d.start()
                ag_sends.append(d)
            out_ref[pl.ds(((my - h) % N_DEV) * CHUNK + s * SEGR, SEGR), :] = (
                ag_recv[h, s * SEGR:(s + 1) * SEGR, :])
    for d in ag_sends:
        d.wait_send()


def kernel(x, Wq, K_ext, V_ext, Wo):
    pos = lax.axis_index("i")
    x2 = _permute_rows(x.reshape(SQ, D_MODEL))
    Ks = _permute_rows(lax.dynamic_slice_in_dim(
        K_ext, pos * HEADS_PER_SHARD, HEADS_PER_SHARD, axis=2
    ).reshape(SKV, HEADS_PER_SHARD * DH))
    Vs = _permute_rows(lax.dynamic_slice_in_dim(
        V_ext, pos * HEADS_PER_SHARD, HEADS_PER_SHARD, axis=2
    ).reshape(SKV, HEADS_PER_SHARD * DH))

    out = pl.pallas_call(
        _body,
        out_shape=jax.ShapeDtypeStruct((SQ, D_MODEL), jnp.float32),
        in_specs=[
            pl.BlockSpec(memory_space=pltpu.VMEM),
            pl.BlockSpec(memory_space=pltpu.VMEM),
            pl.BlockSpec(memory_space=pltpu.MemorySpace.HBM),
            pl.BlockSpec(memory_space=pltpu.MemorySpace.HBM),
            pl.BlockSpec(memory_space=pltpu.VMEM),
        ],
        out_specs=pl.BlockSpec(memory_space=pltpu.VMEM),
        scratch_shapes=[
            pltpu.VMEM((SKV, DH), jnp.float32),
            pltpu.VMEM((SKV, DH), jnp.float32),
            pltpu.VMEM((N_DEV - 1, CHUNK, D_MODEL), jnp.float32),
            pltpu.VMEM((N_DEV - 1, CHUNK, D_MODEL), jnp.float32),
            pltpu.SemaphoreType.DMA((2,)),
            pltpu.SemaphoreType.DMA((N_DEV - 1, SEG)),
            pltpu.SemaphoreType.DMA((N_DEV - 1, SEG)),
            pltpu.SemaphoreType.DMA((N_DEV - 1, SEG)),
            pltpu.SemaphoreType.DMA((N_DEV - 1, SEG)),
        ],
        compiler_params=pltpu.CompilerParams(
            collective_id=0,
            vmem_limit_bytes=63 * 1024 * 1024,
        ),
    )(x2, Wq, Ks, Vs, Wo)
    return _unpermute_rows(out).reshape(1, SQ, D_MODEL)


# device time: 216876 ns/iter; 4.1251x vs baseline; 1.3628x over previous
import numpy as np

import jax
import jax.numpy as jnp
from jax import lax
from jax.experimental import pallas as pl
from jax.experimental.pallas import tpu as pltpu

N_DEV = 8
SQ = 2048
SKV = 2048
D_MODEL = 1024
DH = 128
HEADS_PER_SHARD = 8
CHUNK = SQ // N_DEV
SEG = 4
SEGR = CHUNK // SEG
SCALE = 0.08838834764831843
NEG = -1e9

_blocks = np.arange(SQ // 64)
_n0 = int((_blocks % 3 == 0).sum()) * 64
_n1 = int((_blocks % 3 == 1).sum()) * 64
_n2 = int((_blocks % 3 == 2).sum()) * 64


def _permute_rows(a):
    a3 = a.reshape(SQ // 64, 64, a.shape[-1])
    return jnp.concatenate(
        [a3[0::3], a3[1::3], a3[2::3]], axis=0).reshape(SQ, a.shape[-1])


def _unpermute_rows(a):
    a3 = a.reshape(SQ // 64, 64, a.shape[-1])
    c0, c1, c2 = a3[0:11], a3[11:22], a3[22:32]
    c2p = jnp.concatenate([c2, jnp.zeros_like(a3[:1])], axis=0)
    inter = jnp.stack([c0, c1, c2p], axis=1).reshape(33, 64, a.shape[-1])
    return inter[:32].reshape(SQ, a.shape[-1])
REG = {0: (0, _n0), 1: (_n0, _n1), 2: (_n0 + _n1, _n2)}
CLS = [
    (REG[0], REG[0], None),
    (REG[1], REG[2], REG[1]),
    (REG[2], REG[1], REG[2]),
]


def _body(x_ref, wq_ref, k_hbm, v_hbm, wo_ref, out_ref,
          k_buf, v_buf, rs_recv, ag_recv, rs_sb, ag_sb,
          kv_sems, rs_ssem, rs_rsem, ag_ssem, ag_rsem):
    my = lax.axis_index("i")
    right = (my + 1) % N_DEV
    left = (my + N_DEV - 1) % N_DEV

    barrier_sem = pltpu.get_barrier_semaphore()
    for nbr in (left, right):
        pl.semaphore_signal(barrier_sem, inc=1, device_id=(nbr,),
                            device_id_type=pl.DeviceIdType.MESH)
    pl.semaphore_wait(barrier_sem, 2)

    for j in range(HEADS_PER_SHARD):
        c0, c1 = j * DH, (j + 1) * DH
        ck = pltpu.make_async_copy(k_hbm.at[:, c0:c1], k_buf, kv_sems.at[0])
        cv = pltpu.make_async_copy(v_hbm.at[:, c0:c1], v_buf, kv_sems.at[1])
        ck.start()
        cv.start()
        ck.wait()
        cv.wait()
        wq_j = wq_ref[:, c0:c1]
        wo_j = wo_ref[c0:c1, :]
        for (q0, qn), (a0, an), breg in CLS:
            q = jnp.dot(x_ref[q0:q0 + qn, :], wq_j,
                        preferred_element_type=jnp.float32)
            s_a = lax.dot_general(
                q, k_buf[a0:a0 + an, :], (((1,), (1,)), ((), ())),
                preferred_element_type=jnp.float32) * SCALE
            m_a = jnp.max(s_a, axis=-1, keepdims=True)
            if breg is None:
                w = jnp.exp(s_a - m_a)
                ctx = jnp.dot(w, v_buf[a0:a0 + an, :],
                              preferred_element_type=jnp.float32)
                ctx = ctx / jnp.sum(w, axis=-1, keepdims=True)
            else:
                b0, bn = breg
                s_b = lax.dot_general(
                    q, k_buf[b0:b0 + bn, :], (((1,), (1,)), ((), ())),
                    preferred_element_type=jnp.float32) * SCALE
                rb = lax.broadcasted_iota(jnp.int32, (qn, 1), 0) // 64
                cb = lax.broadcasted_iota(jnp.int32, (1, bn), 1) // 64
                s_b = jnp.where(rb == cb, s_b, NEG)
                s_c = lax.dot_general(
                    q, k_buf[0:64, :], (((1,), (1,)), ((), ())),
                    preferred_element_type=jnp.float32) * SCALE
                m = jnp.maximum(
                    m_a,
                    jnp.maximum(jnp.max(s_b, axis=-1, keepdims=True),
                                jnp.max(s_c, axis=-1, keepdims=True)))
                e_a = jnp.exp(s_a - m)
                e_b = jnp.exp(s_b - m)
                e_c = jnp.exp(s_c - m)
                denom = (jnp.sum(e_a, axis=-1, keepdims=True)
                         + jnp.sum(e_b, axis=-1, keepdims=True)
                         + jnp.sum(e_c, axis=-1, keepdims=True))
                ctx = (jnp.dot(e_a, v_buf[a0:a0 + an, :],
                               preferred_element_type=jnp.float32)
                       + jnp.dot(e_b, v_buf[b0:b0 + bn, :],
                                 preferred_element_type=jnp.float32)
                       + jnp.dot(e_c, v_buf[0:64, :],
                                 preferred_element_type=jnp.float32))
                ctx = ctx / denom
            part = jnp.dot(ctx, wo_j, preferred_element_type=jnp.float32)
            if j == 0:
                out_ref[q0:q0 + qn, :] = part
            else:
                out_ref[q0:q0 + qn, :] = out_ref[q0:q0 + qn, :] + part

    def rs_desc(h, s):
        return pltpu.make_async_remote_copy(
            src_ref=rs_sb.at[h, pl.ds(s * SEGR, SEGR), :],
            dst_ref=rs_recv.at[h, pl.ds(s * SEGR, SEGR), :],
            send_sem=rs_ssem.at[h, s],
            recv_sem=rs_rsem.at[h, s],
            device_id=(right,),
            device_id_type=pl.DeviceIdType.MESH,
        )

    rs_sends = []
    rs_sb[0] = out_ref[pl.ds(my * CHUNK, CHUNK), :].astype(jnp.bfloat16)
    for s in range(SEG):
        d = rs_desc(0, s)
        d.start()
        rs_sends.append(d)
    for h in range(N_DEV - 1):
        recv_c = (my - h - 1) % N_DEV
        for s in range(SEG):
            rs_desc(h, s).wait_recv()
            rows = pl.ds(recv_c * CHUNK + s * SEGR, SEGR)
            acc = (out_ref[rows, :]
                   + rs_recv[h, s * SEGR:(s + 1) * SEGR, :].astype(
                       jnp.float32))
            out_ref[rows, :] = acc
            if h < N_DEV - 2:
                rs_sb[h + 1, s * SEGR:(s + 1) * SEGR, :] = acc.astype(
                    jnp.bfloat16)
                d = rs_desc(h + 1, s)
                d.start()
                rs_sends.append(d)
    for d in rs_sends:
        d.wait_send()

    def ag_desc(h, s):
        if h == 0:
            src = ag_sb.at[pl.ds(s * SEGR, SEGR), :]
        else:
            src = ag_recv.at[h - 1, pl.ds(s * SEGR, SEGR), :]
        return pltpu.make_async_remote_copy(
            src_ref=src,
            dst_ref=ag_recv.at[h, pl.ds(s * SEGR, SEGR), :],
            send_sem=ag_ssem.at[h, s],
            recv_sem=ag_rsem.at[h, s],
            device_id=(right,),
            device_id_type=pl.DeviceIdType.MESH,
        )

    ag_sends = []
    ag_sb[...] = out_ref[
        pl.ds(((my + 1) % N_DEV) * CHUNK, CHUNK), :].astype(jnp.bfloat16)
    for s in range(SEG):
        d = ag_desc(0, s)
        d.start()
        ag_sends.append(d)
    for h in range(N_DEV - 1):
        for s in range(SEG):
            ag_desc(h, s).wait_recv()
            if h < N_DEV - 2:
                d = ag_desc(h + 1, s)
                d.start()
                ag_sends.append(d)
            out_ref[pl.ds(((my - h) % N_DEV) * CHUNK + s * SEGR, SEGR), :] = (
                ag_recv[h, s * SEGR:(s + 1) * SEGR, :].astype(jnp.float32))
    for d in ag_sends:
        d.wait_send()


def kernel(x, Wq, K_ext, V_ext, Wo):
    pos = lax.axis_index("i")
    x2 = _permute_rows(x.reshape(SQ, D_MODEL))
    Ks = _permute_rows(lax.dynamic_slice_in_dim(
        K_ext, pos * HEADS_PER_SHARD, HEADS_PER_SHARD, axis=2
    ).reshape(SKV, HEADS_PER_SHARD * DH))
    Vs = _permute_rows(lax.dynamic_slice_in_dim(
        V_ext, pos * HEADS_PER_SHARD, HEADS_PER_SHARD, axis=2
    ).reshape(SKV, HEADS_PER_SHARD * DH))

    out = pl.pallas_call(
        _body,
        out_shape=jax.ShapeDtypeStruct((SQ, D_MODEL), jnp.float32),
        in_specs=[
            pl.BlockSpec(memory_space=pltpu.VMEM),
            pl.BlockSpec(memory_space=pltpu.VMEM),
            pl.BlockSpec(memory_space=pltpu.MemorySpace.HBM),
            pl.BlockSpec(memory_space=pltpu.MemorySpace.HBM),
            pl.BlockSpec(memory_space=pltpu.VMEM),
        ],
        out_specs=pl.BlockSpec(memory_space=pltpu.VMEM),
        scratch_shapes=[
            pltpu.VMEM((SKV, DH), jnp.float32),
            pltpu.VMEM((SKV, DH), jnp.float32),
            pltpu.VMEM((N_DEV - 1, CHUNK, D_MODEL), jnp.bfloat16),
            pltpu.VMEM((N_DEV - 1, CHUNK, D_MODEL), jnp.bfloat16),
            pltpu.VMEM((N_DEV - 1, CHUNK, D_MODEL), jnp.bfloat16),
            pltpu.VMEM((CHUNK, D_MODEL), jnp.bfloat16),
            pltpu.SemaphoreType.DMA((2,)),
            pltpu.SemaphoreType.DMA((N_DEV - 1, SEG)),
            pltpu.SemaphoreType.DMA((N_DEV - 1, SEG)),
            pltpu.SemaphoreType.DMA((N_DEV - 1, SEG)),
            pltpu.SemaphoreType.DMA((N_DEV - 1, SEG)),
        ],
        compiler_params=pltpu.CompilerParams(
            collective_id=0,
            vmem_limit_bytes=63 * 1024 * 1024,
        ),
    )(x2, Wq, Ks, Vs, Wo)
    return _unpermute_rows(out).reshape(1, SQ, D_MODEL)
